# baseline (speedup 1.0000x reference)
import functools

import jax
import jax.numpy as jnp
from jax import lax
from jax.experimental import pallas as pl
from jax.experimental.pallas import tpu as pltpu
from jax.experimental.pallas import tpu_sc as plsc
import numpy as np

N_NODES = 100000
NODE_DIM = 512
HIDDEN = 256
N_GRAPHS = 512
_MEAN = 0.0
_STDDEV = 1.0
_LOG2 = float(np.log(2.0))


_BLK = 2000


def _mlp_body(x_ref, w1_ref, b1_ref, w2_ref, o_ref):
    xb = x_ref[...].astype(jnp.bfloat16)
    h = jnp.dot(xb, w1_ref[...], preferred_element_type=jnp.float32)
    h = h + b1_ref[...]
    sp = jnp.maximum(h, 0.0) + jnp.log1p(jnp.exp(-jnp.abs(h))) - _LOG2
    o_ref[...] = jnp.dot(sp, w2_ref[...], preferred_element_type=jnp.float32)


def _mlp(x, W1, b1, W2):
    grid = (N_NODES // _BLK,)
    return pl.pallas_call(
        _mlp_body,
        grid=grid,
        in_specs=[
            pl.BlockSpec((_BLK, NODE_DIM), lambda i: (i, 0)),
            pl.BlockSpec((NODE_DIM, HIDDEN), lambda i: (0, 0)),
            pl.BlockSpec((1, HIDDEN), lambda i: (0, 0)),
            pl.BlockSpec((HIDDEN, 1), lambda i: (0, 0)),
        ],
        out_specs=pl.BlockSpec((_BLK, 1), lambda i: (i, 0)),
        out_shape=jax.ShapeDtypeStruct((N_NODES, 1), jnp.float32),
    )(x, W1, b1, W2)


_NS = 16
_PAD = 100096
_CHUNK = _PAD // _NS
_NVEC = _CHUNK // 16


def _seg_body(vals_hbm, idx_hbm, out_hbm, val_v, idx_v, acc2, accv, shared, gath):
    c = lax.axis_index("c")
    s = lax.axis_index("s")
    on = c == 0

    @pl.when(on)
    def _work():
        base = s * _CHUNK
        pltpu.sync_copy(vals_hbm.at[pl.ds(base, _CHUNK)], val_v)
        pltpu.sync_copy(idx_hbm.at[pl.ds(base, _CHUNK)], idx_v)

        def _zero(i, _):
            acc2[pl.ds(i * 16, 16)] = jnp.zeros((16,), jnp.float32)
            return 0

        lax.fori_loop(0, (16 * N_GRAPHS) // 16, _zero, 0)

        lane_off = lax.iota(jnp.int32, 16) * N_GRAPHS

        def _scat(i, _):
            v = val_v[pl.ds(i * 16, 16)]
            ix = idx_v[pl.ds(i * 16, 16)]
            plsc.addupdate_scatter(acc2, [lane_off + ix], v)
            return 0

        lax.fori_loop(0, _NVEC, _scat, 0)

        def _comb(ci, _):
            def _inner(r, t):
                return t + acc2[pl.ds(r * N_GRAPHS + ci * 16, 16)]

            t = lax.fori_loop(0, 16, _inner, jnp.zeros((16,), jnp.float32))
            accv[pl.ds(ci * 16, 16)] = t
            return 0

        lax.fori_loop(0, N_GRAPHS // 16, _comb, 0)
        pltpu.sync_copy(accv, shared.at[s])

    plsc.subcore_barrier()

    @pl.when(jnp.logical_and(on, s == 0))
    def _final():
        pltpu.sync_copy(shared, gath)

        def _fin(ci, _):
            def _inner(r, t):
                return t + gath[r, pl.ds(ci * 16, 16)]

            t = lax.fori_loop(0, _NS, _inner, jnp.zeros((16,), jnp.float32))
            accv[pl.ds(ci * 16, 16)] = t * _STDDEV + _MEAN
            return 0

        lax.fori_loop(0, N_GRAPHS // 16, _fin, 0)
        pltpu.sync_copy(accv, out_hbm)


def _segment_sum(vals, idx):
    mesh = plsc.VectorSubcoreMesh(core_axis_name="c", subcore_axis_name="s")
    f = pl.kernel(
        _seg_body,
        out_type=jax.ShapeDtypeStruct((N_GRAPHS,), jnp.float32),
        mesh=mesh,
        scratch_types=[
            pltpu.VMEM((_CHUNK,), jnp.float32),
            pltpu.VMEM((_CHUNK,), jnp.int32),
            pltpu.VMEM((_NS * N_GRAPHS,), jnp.float32),
            pltpu.VMEM((N_GRAPHS,), jnp.float32),
            pltpu.VMEM_SHARED((_NS, N_GRAPHS), jnp.float32),
            pltpu.VMEM((_NS, N_GRAPHS), jnp.float32),
        ],
        compiler_params=pltpu.CompilerParams(needs_layout_passes=False),
    )
    return f(vals, idx)


@functools.partial(jax.jit)
def kernel(x, W1, b1, W2, batch_idx):
    o = _mlp(x, W1.astype(jnp.bfloat16), b1.reshape(1, HIDDEN), W2)
    vals = jnp.pad(o.reshape(N_NODES), (0, _PAD - N_NODES))
    idx = jnp.pad(batch_idx.astype(jnp.int32), (0, _PAD - N_NODES))
    agg = _segment_sum(vals, idx)
    return agg.reshape(N_GRAPHS, 1)


# speedup vs baseline: 1.0213x; 1.0213x over previous
import functools

import jax
import jax.numpy as jnp
from jax import lax
from jax.experimental import pallas as pl
from jax.experimental.pallas import tpu as pltpu
from jax.experimental.pallas import tpu_sc as plsc
import numpy as np

N_NODES = 100000
NODE_DIM = 512
HIDDEN = 256
N_GRAPHS = 512
_MEAN = 0.0
_STDDEV = 1.0
_LOG2 = float(np.log(2.0))


_BLK = 2000


def _mlp_body(x_ref, w1_ref, b1_ref, w2_ref, o_ref):
    h = jnp.dot(x_ref[...], w1_ref[...], preferred_element_type=jnp.float32)
    h = h + b1_ref[...]
    sp = jnp.maximum(h, 0.0) + jnp.log1p(jnp.exp(-jnp.abs(h))) - _LOG2
    o_ref[...] = jnp.dot(sp, w2_ref[...], preferred_element_type=jnp.float32)


def _mlp(x, W1, b1, W2):
    grid = (N_NODES // _BLK,)
    return pl.pallas_call(
        _mlp_body,
        grid=grid,
        in_specs=[
            pl.BlockSpec((_BLK, NODE_DIM), lambda i: (i, 0)),
            pl.BlockSpec((NODE_DIM, HIDDEN), lambda i: (0, 0)),
            pl.BlockSpec((1, HIDDEN), lambda i: (0, 0)),
            pl.BlockSpec((HIDDEN, 1), lambda i: (0, 0)),
        ],
        out_specs=pl.BlockSpec((_BLK, 1), lambda i: (i, 0)),
        out_shape=jax.ShapeDtypeStruct((N_NODES, 1), jnp.float32),
    )(x, W1, b1, W2)


_NS = 16
_CHUNK = 6256
_TAIL = N_NODES - 15 * _CHUNK


def _seg_body(vals_hbm, idx_hbm, out_hbm, val_v, idx_v, acc2, accv, shared, gath):
    c = lax.axis_index("c")
    s = lax.axis_index("s")
    on = c == 0

    @pl.when(on)
    def _work():
        base = s * _CHUNK

        @pl.when(s < _NS - 1)
        def _full():
            pltpu.sync_copy(vals_hbm.at[pl.ds(base, _CHUNK)], val_v)
            pltpu.sync_copy(idx_hbm.at[pl.ds(base, _CHUNK)], idx_v)

        @pl.when(s == _NS - 1)
        def _tail():
            pltpu.sync_copy(vals_hbm.at[pl.ds(base, _TAIL)], val_v.at[pl.ds(0, _TAIL)])
            pltpu.sync_copy(idx_hbm.at[pl.ds(base, _TAIL)], idx_v.at[pl.ds(0, _TAIL)])

        def _zero(i, _):
            acc2[pl.ds(i * 16, 16)] = jnp.zeros((16,), jnp.float32)
            return 0

        lax.fori_loop(0, (16 * N_GRAPHS) // 16, _zero, 0)

        lane_off = lax.iota(jnp.int32, 16) * N_GRAPHS
        nvec = jnp.where(s == _NS - 1, _TAIL // 16, _CHUNK // 16)

        def _scat(i, _):
            v = val_v[pl.ds(i * 16, 16)]
            ix = idx_v[pl.ds(i * 16, 16)]
            plsc.addupdate_scatter(acc2, [lane_off + ix], v)
            return 0

        lax.fori_loop(0, nvec, _scat, 0)

        def _comb(ci, _):
            def _inner(r, t):
                return t + acc2[pl.ds(r * N_GRAPHS + ci * 16, 16)]

            t = lax.fori_loop(0, 16, _inner, jnp.zeros((16,), jnp.float32))
            accv[pl.ds(ci * 16, 16)] = t
            return 0

        lax.fori_loop(0, N_GRAPHS // 16, _comb, 0)
        pltpu.sync_copy(accv, shared.at[s])

    plsc.subcore_barrier()

    @pl.when(jnp.logical_and(on, s == 0))
    def _final():
        pltpu.sync_copy(shared, gath)

        def _fin(ci, _):
            def _inner(r, t):
                return t + gath[r, pl.ds(ci * 16, 16)]

            t = lax.fori_loop(0, _NS, _inner, jnp.zeros((16,), jnp.float32))
            accv[pl.ds(ci * 16, 16)] = t * _STDDEV + _MEAN
            return 0

        lax.fori_loop(0, N_GRAPHS // 16, _fin, 0)
        pltpu.sync_copy(accv, out_hbm)


def _segment_sum(vals, idx):
    mesh = plsc.VectorSubcoreMesh(core_axis_name="c", subcore_axis_name="s")
    f = pl.kernel(
        _seg_body,
        out_type=jax.ShapeDtypeStruct((N_GRAPHS,), jnp.float32),
        mesh=mesh,
        scratch_types=[
            pltpu.VMEM((_CHUNK,), jnp.float32),
            pltpu.VMEM((_CHUNK,), jnp.int32),
            pltpu.VMEM((_NS * N_GRAPHS,), jnp.float32),
            pltpu.VMEM((N_GRAPHS,), jnp.float32),
            pltpu.VMEM_SHARED((_NS, N_GRAPHS), jnp.float32),
            pltpu.VMEM((_NS, N_GRAPHS), jnp.float32),
        ],
        compiler_params=pltpu.CompilerParams(needs_layout_passes=False),
    )
    return f(vals, idx)


@functools.partial(jax.jit)
def kernel(x, W1, b1, W2, batch_idx):
    o = _mlp(x, W1, b1.reshape(1, HIDDEN), W2)
    agg = _segment_sum(o.reshape(N_NODES), batch_idx.astype(jnp.int32))
    return agg.reshape(N_GRAPHS, 1)


# speedup vs baseline: 1.1452x; 1.1214x over previous
import functools

import jax
import jax.numpy as jnp
from jax import lax
from jax.experimental import pallas as pl
from jax.experimental.pallas import tpu as pltpu
from jax.experimental.pallas import tpu_sc as plsc
import numpy as np

N_NODES = 100000
NODE_DIM = 512
HIDDEN = 256
N_GRAPHS = 512
_MEAN = 0.0
_STDDEV = 1.0
_LOG2 = float(np.log(2.0))


_BLK = 4000


def _mlp_body(x_ref, w1_ref, b1_ref, w2_ref, o_ref):
    h = jnp.dot(x_ref[...], w1_ref[...], preferred_element_type=jnp.float32)
    h = h + b1_ref[...]
    sp = jnp.maximum(h, 0.0) + jnp.log1p(jnp.exp(-jnp.abs(h))) - _LOG2
    o_ref[...] = jnp.dot(sp, w2_ref[...], preferred_element_type=jnp.float32)


def _mlp(x, W1, b1, W2):
    grid = (N_NODES // _BLK,)
    return pl.pallas_call(
        _mlp_body,
        grid=grid,
        in_specs=[
            pl.BlockSpec((_BLK, NODE_DIM), lambda i: (i, 0)),
            pl.BlockSpec((NODE_DIM, HIDDEN), lambda i: (0, 0)),
            pl.BlockSpec((1, HIDDEN), lambda i: (0, 0)),
            pl.BlockSpec((HIDDEN, 1), lambda i: (0, 0)),
        ],
        out_specs=pl.BlockSpec((_BLK, 1), lambda i: (i, 0)),
        out_shape=jax.ShapeDtypeStruct((N_NODES, 1), jnp.float32),
    )(x, W1, b1, W2)


_NS = 16
_CHUNK = 6256
_TAIL = N_NODES - 15 * _CHUNK


def _seg_body(vals_hbm, idx_hbm, out_hbm, val_v, idx_v, acc2, accv, shared, gath):
    c = lax.axis_index("c")
    s = lax.axis_index("s")
    on = c == 0

    @pl.when(on)
    def _work():
        base = s * _CHUNK

        @pl.when(s < _NS - 1)
        def _full():
            pltpu.sync_copy(vals_hbm.at[pl.ds(base, _CHUNK)], val_v)
            pltpu.sync_copy(idx_hbm.at[pl.ds(base, _CHUNK)], idx_v)

        @pl.when(s == _NS - 1)
        def _tail():
            pltpu.sync_copy(vals_hbm.at[pl.ds(base, _TAIL)], val_v.at[pl.ds(0, _TAIL)])
            pltpu.sync_copy(idx_hbm.at[pl.ds(base, _TAIL)], idx_v.at[pl.ds(0, _TAIL)])

        def _zero(i, _):
            acc2[pl.ds(i * 16, 16)] = jnp.zeros((16,), jnp.float32)
            return 0

        lax.fori_loop(0, (16 * N_GRAPHS) // 16, _zero, 0)

        lane_off = lax.iota(jnp.int32, 16) * N_GRAPHS
        nvec = jnp.where(s == _NS - 1, _TAIL // 16, _CHUNK // 16)

        def _scat(i, _):
            v = val_v[pl.ds(i * 16, 16)]
            ix = idx_v[pl.ds(i * 16, 16)]
            plsc.addupdate_scatter(acc2, [lane_off + ix], v)
            return 0

        lax.fori_loop(0, nvec, _scat, 0)

        def _comb(ci, _):
            def _inner(r, t):
                return t + acc2[pl.ds(r * N_GRAPHS + ci * 16, 16)]

            t = lax.fori_loop(0, 16, _inner, jnp.zeros((16,), jnp.float32))
            accv[pl.ds(ci * 16, 16)] = t
            return 0

        lax.fori_loop(0, N_GRAPHS // 16, _comb, 0)
        pltpu.sync_copy(accv, shared.at[s])

    plsc.subcore_barrier()

    @pl.when(jnp.logical_and(on, s == 0))
    def _final():
        pltpu.sync_copy(shared, gath)

        def _fin(ci, _):
            def _inner(r, t):
                return t + gath[r, pl.ds(ci * 16, 16)]

            t = lax.fori_loop(0, _NS, _inner, jnp.zeros((16,), jnp.float32))
            accv[pl.ds(ci * 16, 16)] = t * _STDDEV + _MEAN
            return 0

        lax.fori_loop(0, N_GRAPHS // 16, _fin, 0)
        pltpu.sync_copy(accv, out_hbm)


def _segment_sum(vals, idx):
    mesh = plsc.VectorSubcoreMesh(core_axis_name="c", subcore_axis_name="s")
    f = pl.kernel(
        _seg_body,
        out_type=jax.ShapeDtypeStruct((N_GRAPHS,), jnp.float32),
        mesh=mesh,
        scratch_types=[
            pltpu.VMEM((_CHUNK,), jnp.float32),
            pltpu.VMEM((_CHUNK,), jnp.int32),
            pltpu.VMEM((_NS * N_GRAPHS,), jnp.float32),
            pltpu.VMEM((N_GRAPHS,), jnp.float32),
            pltpu.VMEM_SHARED((_NS, N_GRAPHS), jnp.float32),
            pltpu.VMEM((_NS, N_GRAPHS), jnp.float32),
        ],
        compiler_params=pltpu.CompilerParams(needs_layout_passes=False),
    )
    return f(vals, idx)


@functools.partial(jax.jit)
def kernel(x, W1, b1, W2, batch_idx):
    o = _mlp(x, W1, b1.reshape(1, HIDDEN), W2)
    agg = _segment_sum(o.reshape(N_NODES), batch_idx.astype(jnp.int32))
    return agg.reshape(N_GRAPHS, 1)


# speedup vs baseline: 1.2056x; 1.0527x over previous
import functools

import jax
import jax.numpy as jnp
from jax import lax
from jax.experimental import pallas as pl
from jax.experimental.pallas import tpu as pltpu
from jax.experimental.pallas import tpu_sc as plsc
import numpy as np

N_NODES = 100000
NODE_DIM = 512
HIDDEN = 256
N_GRAPHS = 512
_MEAN = 0.0
_STDDEV = 1.0
_LOG2 = float(np.log(2.0))


_BLK = 10000


def _mlp_body(x_ref, w1_ref, b1_ref, w2_ref, o_ref):
    h = jnp.dot(x_ref[...], w1_ref[...], preferred_element_type=jnp.float32)
    h = h + b1_ref[...]
    sp = jnp.maximum(h, 0.0) + jnp.log1p(jnp.exp(-jnp.abs(h))) - _LOG2
    o_ref[...] = jnp.dot(sp, w2_ref[...], preferred_element_type=jnp.float32)


def _mlp(x, W1, b1, W2):
    grid = (N_NODES // _BLK,)
    return pl.pallas_call(
        _mlp_body,
        grid=grid,
        in_specs=[
            pl.BlockSpec((_BLK, NODE_DIM), lambda i: (i, 0)),
            pl.BlockSpec((NODE_DIM, HIDDEN), lambda i: (0, 0)),
            pl.BlockSpec((1, HIDDEN), lambda i: (0, 0)),
            pl.BlockSpec((HIDDEN, 1), lambda i: (0, 0)),
        ],
        out_specs=pl.BlockSpec((_BLK, 1), lambda i: (i, 0)),
        out_shape=jax.ShapeDtypeStruct((N_NODES, 1), jnp.float32),
    )(x, W1, b1, W2)


_NS = 16
_CHUNK = 6256
_TAIL = N_NODES - 15 * _CHUNK


def _seg_body(vals_hbm, idx_hbm, out_hbm, val_v, idx_v, acc2, accv, shared, gath):
    c = lax.axis_index("c")
    s = lax.axis_index("s")
    on = c == 0

    @pl.when(on)
    def _work():
        base = s * _CHUNK

        @pl.when(s < _NS - 1)
        def _full():
            pltpu.sync_copy(vals_hbm.at[pl.ds(base, _CHUNK)], val_v)
            pltpu.sync_copy(idx_hbm.at[pl.ds(base, _CHUNK)], idx_v)

        @pl.when(s == _NS - 1)
        def _tail():
            pltpu.sync_copy(vals_hbm.at[pl.ds(base, _TAIL)], val_v.at[pl.ds(0, _TAIL)])
            pltpu.sync_copy(idx_hbm.at[pl.ds(base, _TAIL)], idx_v.at[pl.ds(0, _TAIL)])

        def _zero(i, _):
            acc2[pl.ds(i * 16, 16)] = jnp.zeros((16,), jnp.float32)
            return 0

        lax.fori_loop(0, (16 * N_GRAPHS) // 16, _zero, 0)

        lane_off = lax.iota(jnp.int32, 16) * N_GRAPHS
        nvec = jnp.where(s == _NS - 1, _TAIL // 16, _CHUNK // 16)

        def _scat(i, _):
            v = val_v[pl.ds(i * 16, 16)]
            ix = idx_v[pl.ds(i * 16, 16)]
            plsc.addupdate_scatter(acc2, [lane_off + ix], v)
            return 0

        lax.fori_loop(0, nvec, _scat, 0)

        def _comb(ci, _):
            def _inner(r, t):
                return t + acc2[pl.ds(r * N_GRAPHS + ci * 16, 16)]

            t = lax.fori_loop(0, 16, _inner, jnp.zeros((16,), jnp.float32))
            accv[pl.ds(ci * 16, 16)] = t
            return 0

        lax.fori_loop(0, N_GRAPHS // 16, _comb, 0)
        pltpu.sync_copy(accv, shared.at[s])

    plsc.subcore_barrier()

    @pl.when(jnp.logical_and(on, s == 0))
    def _final():
        pltpu.sync_copy(shared, gath)

        def _fin(ci, _):
            def _inner(r, t):
                return t + gath[r, pl.ds(ci * 16, 16)]

            t = lax.fori_loop(0, _NS, _inner, jnp.zeros((16,), jnp.float32))
            accv[pl.ds(ci * 16, 16)] = t * _STDDEV + _MEAN
            return 0

        lax.fori_loop(0, N_GRAPHS // 16, _fin, 0)
        pltpu.sync_copy(accv, out_hbm)


def _segment_sum(vals, idx):
    mesh = plsc.VectorSubcoreMesh(core_axis_name="c", subcore_axis_name="s")
    f = pl.kernel(
        _seg_body,
        out_type=jax.ShapeDtypeStruct((N_GRAPHS,), jnp.float32),
        mesh=mesh,
        scratch_types=[
            pltpu.VMEM((_CHUNK,), jnp.float32),
            pltpu.VMEM((_CHUNK,), jnp.int32),
            pltpu.VMEM((_NS * N_GRAPHS,), jnp.float32),
            pltpu.VMEM((N_GRAPHS,), jnp.float32),
            pltpu.VMEM_SHARED((_NS, N_GRAPHS), jnp.float32),
            pltpu.VMEM((_NS, N_GRAPHS), jnp.float32),
        ],
        compiler_params=pltpu.CompilerParams(needs_layout_passes=False),
    )
    return f(vals, idx)


@functools.partial(jax.jit)
def kernel(x, W1, b1, W2, batch_idx):
    o = _mlp(x, W1, b1.reshape(1, HIDDEN), W2)
    agg = _segment_sum(o.reshape(N_NODES), batch_idx.astype(jnp.int32))
    return agg.reshape(N_GRAPHS, 1)


# speedup vs baseline: 1.2452x; 1.0329x over previous
import functools

import jax
import jax.numpy as jnp
from jax import lax
from jax.experimental import pallas as pl
from jax.experimental.pallas import tpu as pltpu
from jax.experimental.pallas import tpu_sc as plsc
import numpy as np

N_NODES = 100000
NODE_DIM = 512
HIDDEN = 256
N_GRAPHS = 512
_MEAN = 0.0
_STDDEV = 1.0
_LOG2 = float(np.log(2.0))


_BLK = 10000


def _mlp_body(x_ref, w1_ref, b1_ref, w2_ref, o_ref):
    h = jnp.dot(x_ref[...], w1_ref[...], preferred_element_type=jnp.float32)
    h = h + b1_ref[...]
    sp = jnp.maximum(h, 0.0) + jnp.log1p(jnp.exp(-jnp.abs(h))) - _LOG2
    o_ref[...] = jnp.dot(sp, w2_ref[...], preferred_element_type=jnp.float32)


def _mlp(x, W1, b1, W2):
    grid = (N_NODES // _BLK,)
    return pl.pallas_call(
        _mlp_body,
        grid=grid,
        in_specs=[
            pl.BlockSpec((_BLK, NODE_DIM), lambda i: (i, 0)),
            pl.BlockSpec((NODE_DIM, HIDDEN), lambda i: (0, 0)),
            pl.BlockSpec((1, HIDDEN), lambda i: (0, 0)),
            pl.BlockSpec((HIDDEN, 1), lambda i: (0, 0)),
        ],
        out_specs=pl.BlockSpec((_BLK, 1), lambda i: (i, 0)),
        out_shape=jax.ShapeDtypeStruct((N_NODES, 1), jnp.float32),
    )(x, W1, b1, W2)


_NS = 16
_CHUNK = 6256
_TAIL = N_NODES - 15 * _CHUNK


_COLS = N_GRAPHS // _NS


def _seg_body(vals_hbm, idx_hbm, out_hbm, val_v, idx_v, acc2, accv, shared, gath, sem):
    c = lax.axis_index("c")
    s = lax.axis_index("s")
    on = c == 0

    @pl.when(on)
    def _work():
        base = s * _CHUNK

        @pl.when(s < _NS - 1)
        def _full():
            cp_v = pltpu.async_copy(vals_hbm.at[pl.ds(base, _CHUNK)], val_v, sem)
            cp_i = pltpu.async_copy(idx_hbm.at[pl.ds(base, _CHUNK)], idx_v, sem)
            cp_v.wait()
            cp_i.wait()

        @pl.when(s == _NS - 1)
        def _tail():
            cp_v = pltpu.async_copy(
                vals_hbm.at[pl.ds(base, _TAIL)], val_v.at[pl.ds(0, _TAIL)], sem)
            cp_i = pltpu.async_copy(
                idx_hbm.at[pl.ds(base, _TAIL)], idx_v.at[pl.ds(0, _TAIL)], sem)
            cp_v.wait()
            cp_i.wait()

        zero16 = jnp.zeros((16,), jnp.float32)

        def _zero(i, _):
            for k in range(8):
                acc2[pl.ds((i * 8 + k) * 16, 16)] = zero16
            return 0

        lax.fori_loop(0, (16 * N_GRAPHS) // (16 * 8), _zero, 0)

        lane_off = lax.iota(jnp.int32, 16) * N_GRAPHS
        nvec = jnp.where(s == _NS - 1, _TAIL // 16, _CHUNK // 16)
        n4 = nvec // 4

        def _scat4(i, _):
            for k in range(4):
                j = i * 4 + k
                v = val_v[pl.ds(j * 16, 16)]
                ix = idx_v[pl.ds(j * 16, 16)]
                plsc.addupdate_scatter(acc2, [lane_off + ix], v)
            return 0

        lax.fori_loop(0, n4, _scat4, 0)

        def _scat1(j, _):
            v = val_v[pl.ds(j * 16, 16)]
            ix = idx_v[pl.ds(j * 16, 16)]
            plsc.addupdate_scatter(acc2, [lane_off + ix], v)
            return 0

        lax.fori_loop(n4 * 4, nvec, _scat1, 0)

        def _comb(ci, _):
            t = zero16
            for r in range(16):
                t = t + acc2[pl.ds(r * N_GRAPHS + ci * 16, 16)]
            accv[pl.ds(ci * 16, 16)] = t
            return 0

        lax.fori_loop(0, N_GRAPHS // 16, _comb, 0)
        pltpu.sync_copy(accv, shared.at[s])

    plsc.subcore_barrier()

    @pl.when(on)
    def _final():
        pltpu.sync_copy(shared, gath)
        for ci in range(_COLS // 16):
            t = jnp.zeros((16,), jnp.float32)
            for r in range(_NS):
                t = t + gath[r, pl.ds(s * _COLS + ci * 16, 16)]
            accv[pl.ds(ci * 16, 16)] = t * _STDDEV + _MEAN
        pltpu.sync_copy(accv.at[pl.ds(0, _COLS)], out_hbm.at[pl.ds(s * _COLS, _COLS)])


def _segment_sum(vals, idx):
    mesh = plsc.VectorSubcoreMesh(core_axis_name="c", subcore_axis_name="s")
    f = pl.kernel(
        _seg_body,
        out_type=jax.ShapeDtypeStruct((N_GRAPHS,), jnp.float32),
        mesh=mesh,
        scratch_types=[
            pltpu.VMEM((_CHUNK,), jnp.float32),
            pltpu.VMEM((_CHUNK,), jnp.int32),
            pltpu.VMEM((_NS * N_GRAPHS,), jnp.float32),
            pltpu.VMEM((N_GRAPHS,), jnp.float32),
            pltpu.VMEM_SHARED((_NS, N_GRAPHS), jnp.float32),
            pltpu.VMEM((_NS, N_GRAPHS), jnp.float32),
            pltpu.SemaphoreType.DMA,
        ],
        compiler_params=pltpu.CompilerParams(needs_layout_passes=False),
    )
    return f(vals, idx)


@functools.partial(jax.jit)
def kernel(x, W1, b1, W2, batch_idx):
    o = _mlp(x, W1, b1.reshape(1, HIDDEN), W2)
    agg = _segment_sum(o.reshape(N_NODES), batch_idx.astype(jnp.int32))
    return agg.reshape(N_GRAPHS, 1)
